# baseline (speedup 1.0000x reference)
import functools

import jax
import jax.numpy as jnp
from jax.experimental import pallas as pl

TOKENS = 32768
DIM = 768
N_EXPERTS = 8
BLOCK = 4096


def _top1(s):
    st = s.T
    m = jnp.max(st, axis=0, keepdims=True)
    denom = jnp.sum(jnp.exp(st - m), axis=0, keepdims=True)
    return ((1.0 / denom).reshape(BLOCK),
            jnp.argmax(st, axis=0).reshape(BLOCK).astype(jnp.int32))


def _gate_kernel(xa_ref, xb_ref, w_ref, w_out_ref, idx_out_ref):
    w = w_ref[...]
    for j, x_ref in enumerate((xa_ref, xb_ref)):
        s = jax.lax.dot_general(
            x_ref[...], w,
            dimension_numbers=(((1,), (1,)), ((), ())),
            preferred_element_type=jnp.float32)
        wv, iv = _top1(s)
        w_out_ref[pl.ds(j * BLOCK, BLOCK)] = wv
        idx_out_ref[pl.ds(j * BLOCK, BLOCK)] = iv


@jax.jit
def kernel(x, weight):
    grid = (TOKENS // (2 * BLOCK),)
    weights, indices = pl.pallas_call(
        _gate_kernel,
        grid=grid,
        in_specs=[
            pl.BlockSpec((BLOCK, DIM), lambda i: (2 * i, 0)),
            pl.BlockSpec((BLOCK, DIM), lambda i: (2 * i + 1, 0)),
            pl.BlockSpec((N_EXPERTS, DIM), lambda i: (0, 0)),
        ],
        out_specs=[
            pl.BlockSpec((2 * BLOCK,), lambda i: (i,)),
            pl.BlockSpec((2 * BLOCK,), lambda i: (i,)),
        ],
        out_shape=[
            jax.ShapeDtypeStruct((TOKENS,), jnp.float32),
            jax.ShapeDtypeStruct((TOKENS,), jnp.int32),
        ],
    )(x, x, weight)
    return weights.reshape(TOKENS, 1), indices.reshape(TOKENS, 1)


# speedup vs baseline: 1.0758x; 1.0758x over previous
import functools

import jax
import jax.numpy as jnp
from jax.experimental import pallas as pl

TOKENS = 32768
DIM = 768
N_EXPERTS = 8
BLOCK = 4096


def _gate_kernel(x_ref, w_ref, w_out_ref, idx_out_ref):
    s = jax.lax.dot_general(
        x_ref[...], w_ref[...],
        dimension_numbers=(((1,), (1,)), ((), ())),
        preferred_element_type=jnp.float32)
    st = s.T
    m = jnp.max(st, axis=0, keepdims=True)
    denom = jnp.sum(jnp.exp(st - m), axis=0, keepdims=True)
    w_out_ref[...] = (1.0 / denom).reshape(BLOCK)
    idx_out_ref[...] = jnp.argmax(st, axis=0).reshape(BLOCK).astype(jnp.int32)


@jax.jit
def kernel(x, weight):
    grid = (TOKENS // BLOCK,)
    weights, indices = pl.pallas_call(
        _gate_kernel,
        grid=grid,
        in_specs=[
            pl.BlockSpec((BLOCK, DIM), lambda i: (i, 0)),
            pl.BlockSpec((N_EXPERTS, DIM), lambda i: (0, 0)),
        ],
        out_specs=[
            pl.BlockSpec((BLOCK,), lambda i: (i,)),
            pl.BlockSpec((BLOCK,), lambda i: (i,)),
        ],
        out_shape=[
            jax.ShapeDtypeStruct((TOKENS,), jnp.float32),
            jax.ShapeDtypeStruct((TOKENS,), jnp.int32),
        ],
    )(x, weight)
    return weights.reshape(TOKENS, 1), indices.reshape(TOKENS, 1)
